# baseline (speedup 1.0000x reference)
import functools

import jax
import jax.numpy as jnp
from jax.experimental import pallas as pl

_COMMIT = 0.25
_WIN = 2048


def _vq_body(zf_ref, w_ref, zq_ref, idx_ref, loss_ref):
    z = zf_ref[...]
    w = w_ref[...]
    t, d = z.shape
    e = w.shape[0]

    zsq = jnp.sum(z * z, axis=1, keepdims=True)
    ones_row = jnp.ones((8, d), jnp.float32)
    esq = jax.lax.dot_general(
        ones_row, w * w, (((1,), (1,)), ((), ())),
        precision=jax.lax.Precision.HIGHEST,
        preferred_element_type=jnp.float32)[0:1, :]

    m = jax.lax.dot_general(
        z.astype(jnp.bfloat16), w, (((1,), (1,)), ((), ())),
        preferred_element_type=jnp.float32)

    dist = (zsq + esq) - 2.0 * m

    lanes = jax.lax.broadcasted_iota(jnp.int32, (t, _WIN), 1)
    nwin = e // _WIN
    winner = jnp.zeros((t,), jnp.int32)
    run = jnp.zeros((t,), jnp.float32)
    for k in range(nwin):
        dk = dist[:, k * _WIN:(k + 1) * _WIN]
        mk = jnp.min(dk, axis=1, keepdims=True)
        ak = jnp.min(jnp.where(dk == mk, lanes, _WIN), axis=1) + k * _WIN
        mk = mk[:, 0]
        rk = mk.astype(jnp.bfloat16).astype(jnp.float32)
        if k == 0:
            winner = ak
            run = rk
        else:
            take = mk < run
            winner = jnp.where(take, ak, winner)
            run = jnp.where(take, rk, run)

    oh = (jax.lax.broadcasted_iota(jnp.int32, (t, e), 1)
          == winner[:, None]).astype(jnp.float32)
    zq = jax.lax.dot_general(
        oh, w, (((1,), (0,)), ((), ())),
        precision=jax.lax.Precision.HIGHEST,
        preferred_element_type=jnp.float32)

    zq_ref[...] = zq
    idx_ref[...] = winner.reshape(1, 1, t)

    diff = zq - z
    part = jnp.sum(diff * diff, keepdims=True).reshape(1, 1)
    step = pl.program_id(0)
    prev = jnp.where(step == 0, jnp.zeros((1, 1), jnp.float32), loss_ref[...])
    loss_ref[...] = prev + part


@functools.partial(jax.jit, static_argnames=())
def kernel(z, embedding_weight):
    b, s, d = z.shape
    n = b * s
    e = embedding_weight.shape[0]
    tile = 128
    g = n // tile
    zf = z.reshape(n, d)

    zq_out, idx_out, loss_out = pl.pallas_call(
        _vq_body,
        grid=(g,),
        in_specs=[
            pl.BlockSpec((tile, d), lambda i: (i, 0)),
            pl.BlockSpec((e, d), lambda i: (0, 0)),
        ],
        out_specs=[
            pl.BlockSpec((tile, d), lambda i: (i, 0)),
            pl.BlockSpec((1, 1, tile), lambda i: (i, 0, 0)),
            pl.BlockSpec((1, 1), lambda i: (0, 0)),
        ],
        out_shape=[
            jax.ShapeDtypeStruct((n, d), jnp.float32),
            jax.ShapeDtypeStruct((g, 1, tile), jnp.int32),
            jax.ShapeDtypeStruct((1, 1), jnp.float32),
        ],
    )(zf, embedding_weight)

    z_q_st = zq_out.reshape(b, s, d)
    indices = idx_out.reshape(b, s)
    mean = loss_out[0, 0] / jnp.float32(n * d)
    loss = mean + _COMMIT * mean
    return z_q_st, loss, indices


# speedup vs baseline: 2.4184x; 2.4184x over previous
import functools

import jax
import jax.numpy as jnp
from jax.experimental import pallas as pl
from jax.experimental.pallas import tpu as pltpu

_COMMIT = 0.25
_WIN = 2048


def _vq_body(zf_ref, w_ref, zq_ref, idx_ref, loss_ref,
             esq_ref, whi_ref, wlo_ref):
    step = pl.program_id(0)
    z = zf_ref[...]
    w = w_ref[...]
    t, d = z.shape
    e = w.shape[0]

    @pl.when(step == 0)
    def _prep():
        ones_row = jnp.ones((8, d), jnp.float32)
        esq_ref[...] = jax.lax.dot_general(
            ones_row, w * w, (((1,), (1,)), ((), ())),
            precision=jax.lax.Precision.HIGHEST,
            preferred_element_type=jnp.float32)[0:1, :]
        hi = w.astype(jnp.bfloat16)
        whi_ref[...] = hi
        wlo_ref[...] = (w - hi.astype(jnp.float32)).astype(jnp.bfloat16)

    zsq = jnp.sum(z * z, axis=1, keepdims=True)

    m = jax.lax.dot_general(
        z.astype(jnp.bfloat16), w, (((1,), (1,)), ((), ())),
        preferred_element_type=jnp.float32)

    dist = (zsq + esq_ref[...]) - 2.0 * m

    nwin = e // _WIN
    mins = [jnp.min(dist[:, k * _WIN:(k + 1) * _WIN], axis=1, keepdims=True)
            for k in range(nwin)]
    winwid = jnp.zeros((t, 1), jnp.int32)
    winval = mins[0]
    run = mins[0].astype(jnp.bfloat16).astype(jnp.float32)
    for k in range(1, nwin):
        take = mins[k] < run
        winwid = jnp.where(take, k, winwid)
        winval = jnp.where(take, mins[k], winval)
        run = jnp.where(take, mins[k].astype(jnp.bfloat16).astype(jnp.float32),
                        run)

    lanes = jax.lax.broadcasted_iota(jnp.int32, (t, e), 1)
    hit = (dist == winval) & ((lanes >> 11) == winwid)
    idx = jnp.min(jnp.where(hit, lanes, e), axis=1)

    oh = (lanes == idx[:, None]).astype(jnp.bfloat16)
    zq = (jax.lax.dot_general(oh, whi_ref[...], (((1,), (0,)), ((), ())),
                              preferred_element_type=jnp.float32)
          + jax.lax.dot_general(oh, wlo_ref[...], (((1,), (0,)), ((), ())),
                                preferred_element_type=jnp.float32))

    zq_ref[...] = zq
    idx_ref[...] = idx.reshape(1, 1, t)

    diff = zq - z
    part = jnp.sum(diff * diff, keepdims=True).reshape(1, 1)
    prev = jnp.where(step == 0, jnp.zeros((1, 1), jnp.float32), loss_ref[...])
    loss_ref[...] = prev + part


@functools.partial(jax.jit, static_argnames=())
def kernel(z, embedding_weight):
    b, s, d = z.shape
    n = b * s
    e = embedding_weight.shape[0]
    tile = 256
    g = n // tile
    zf = z.reshape(n, d)

    zq_out, idx_out, loss_out = pl.pallas_call(
        _vq_body,
        grid=(g,),
        in_specs=[
            pl.BlockSpec((tile, d), lambda i: (i, 0)),
            pl.BlockSpec((e, d), lambda i: (0, 0)),
        ],
        out_specs=[
            pl.BlockSpec((tile, d), lambda i: (i, 0)),
            pl.BlockSpec((1, 1, tile), lambda i: (i, 0, 0)),
            pl.BlockSpec((1, 1), lambda i: (0, 0)),
        ],
        out_shape=[
            jax.ShapeDtypeStruct((n, d), jnp.float32),
            jax.ShapeDtypeStruct((g, 1, tile), jnp.int32),
            jax.ShapeDtypeStruct((1, 1), jnp.float32),
        ],
        scratch_shapes=[
            pltpu.VMEM((1, e), jnp.float32),
            pltpu.VMEM((e, d), jnp.bfloat16),
            pltpu.VMEM((e, d), jnp.bfloat16),
        ],
    )(zf, embedding_weight)

    z_q_st = zq_out.reshape(b, s, d)
    indices = idx_out.reshape(b, s)
    mean = loss_out[0, 0] / jnp.float32(n * d)
    loss = mean + _COMMIT * mean
    return z_q_st, loss, indices


# speedup vs baseline: 2.8912x; 1.1955x over previous
import functools

import jax
import jax.numpy as jnp
from jax.experimental import pallas as pl
from jax.experimental.pallas import tpu as pltpu
from jax.experimental.pallas import tpu_sc as plsc

_COMMIT = 0.25
_WIN = 2048


def _vq_idx_body(zf_ref, w_ref, idx_ref, loss_ref, esq_ref):
    step = pl.program_id(0)
    z = zf_ref[...]
    w = w_ref[...]
    t, d = z.shape
    e = w.shape[0]

    @pl.when(step == 0)
    def _prep():
        ones_row = jnp.ones((8, d), jnp.float32)
        esq_ref[...] = jax.lax.dot_general(
            ones_row, w * w, (((1,), (1,)), ((), ())),
            precision=jax.lax.Precision.HIGHEST,
            preferred_element_type=jnp.float32)[0:1, :]

    zsq = jnp.sum(z * z, axis=1, keepdims=True)
    m = jax.lax.dot_general(
        z.astype(jnp.bfloat16), w, (((1,), (1,)), ((), ())),
        preferred_element_type=jnp.float32)
    dist = (zsq + esq_ref[...]) - 2.0 * m

    nwin = e // _WIN
    mins = [jnp.min(dist[:, k * _WIN:(k + 1) * _WIN], axis=1, keepdims=True)
            for k in range(nwin)]
    winwid = jnp.zeros((t, 1), jnp.int32)
    winval = mins[0]
    run = mins[0].astype(jnp.bfloat16).astype(jnp.float32)
    for k in range(1, nwin):
        take = mins[k] < run
        winwid = jnp.where(take, k, winwid)
        winval = jnp.where(take, mins[k], winval)
        run = jnp.where(take, mins[k].astype(jnp.bfloat16).astype(jnp.float32),
                        run)

    lanes = jax.lax.broadcasted_iota(jnp.int32, (t, e), 1)
    hit = (dist == winval) & ((lanes >> 11) == winwid)
    idx = jnp.min(jnp.where(hit, lanes, e), axis=1)
    idx_ref[...] = idx.reshape(1, 1, t)

    part = jnp.sum(winval, keepdims=True).reshape(1, 1)
    prev = jnp.where(step == 0, jnp.zeros((1, 1), jnp.float32), loss_ref[...])
    loss_ref[...] = prev + part


def _sc_gather_grouped(table128, idx4):
    n = idx4.shape[0]
    d = table128.shape[1]
    num_cores, num_subcores = 2, 16
    nw = num_cores * num_subcores
    b_per_w = n // nw
    chunk = 128
    nchunk = b_per_w // chunk
    mesh = plsc.VectorSubcoreMesh(core_axis_name="c", subcore_axis_name="s",
                                  num_cores=num_cores,
                                  num_subcores=num_subcores)

    @functools.partial(
        pl.kernel, mesh=mesh,
        out_type=jax.ShapeDtypeStruct((n, d), jnp.float32),
        scratch_types=[
            pltpu.VMEM((chunk,), jnp.int32),
            pltpu.VMEM((chunk, d), jnp.float32),
            pltpu.SemaphoreType.DMA,
        ],
    )
    def k(table_hbm, idx_hbm, out_hbm, idx_v, rows_v, sem):
        wid = jax.lax.axis_index("s") * num_cores + jax.lax.axis_index("c")
        base = wid * b_per_w
        for c in range(nchunk):
            off = base + c * chunk
            pltpu.sync_copy(idx_hbm.at[pl.ds(off, chunk)], idx_v)
            pltpu.async_copy(table_hbm.at[idx_v], rows_v, sem).wait()
            pltpu.sync_copy(rows_v, out_hbm.at[pl.ds(off, chunk)])

    return k(table128, idx4)


def _select_body(rows_ref, idx_ref, out_ref):
    rows = rows_ref[...]
    sub = idx_ref[...].reshape(-1, 1) & 3
    d = out_ref.shape[1]
    acc = jnp.where(sub == 0, rows[:, 0:d], 0.0)
    for q in range(1, 4):
        acc = jnp.where(sub == q, rows[:, q * d:(q + 1) * d], acc)
    out_ref[...] = acc


@functools.partial(jax.jit, static_argnames=())
def kernel(z, embedding_weight):
    b, s, d = z.shape
    n = b * s
    e = embedding_weight.shape[0]
    tile = 256
    g = n // tile
    zf = z.reshape(n, d)

    idx_out, loss_out = pl.pallas_call(
        _vq_idx_body,
        grid=(g,),
        in_specs=[
            pl.BlockSpec((tile, d), lambda i: (i, 0)),
            pl.BlockSpec((e, d), lambda i: (0, 0)),
        ],
        out_specs=[
            pl.BlockSpec((1, 1, tile), lambda i: (i, 0, 0)),
            pl.BlockSpec((1, 1), lambda i: (0, 0)),
        ],
        out_shape=[
            jax.ShapeDtypeStruct((g, 1, tile), jnp.int32),
            jax.ShapeDtypeStruct((1, 1), jnp.float32),
        ],
        scratch_shapes=[
            pltpu.VMEM((1, e), jnp.float32),
        ],
    )(zf, embedding_weight)

    indices_flat = idx_out.reshape(n)
    rows128 = _sc_gather_grouped(embedding_weight.reshape(e // 4, 4 * d),
                                 indices_flat >> 2)

    sel_tile = 1024
    g2 = n // sel_tile
    zq = pl.pallas_call(
        _select_body,
        grid=(g2,),
        in_specs=[
            pl.BlockSpec((sel_tile, 4 * d), lambda i: (i, 0)),
            pl.BlockSpec((1, 1, sel_tile), lambda i: (i, 0, 0)),
        ],
        out_specs=pl.BlockSpec((sel_tile, d), lambda i: (i, 0)),
        out_shape=jax.ShapeDtypeStruct((n, d), jnp.float32),
    )(rows128, indices_flat.reshape(g2, 1, sel_tile))

    z_q_st = zq.reshape(b, s, d)
    indices = indices_flat.reshape(b, s)
    mean = loss_out[0, 0] / jnp.float32(n * d)
    loss = mean + _COMMIT * mean
    return z_q_st, loss, indices


# speedup vs baseline: 3.5318x; 1.2216x over previous
import functools

import jax
import jax.numpy as jnp
from jax.experimental import pallas as pl
from jax.experimental.pallas import tpu as pltpu
from jax.experimental.pallas import tpu_sc as plsc

_COMMIT = 0.25
_WIN = 2048


def _vq_idx_body(zf_ref, w_ref, idx_ref, loss_ref, esq_ref):
    step = pl.program_id(0)
    z = zf_ref[...]
    w = w_ref[...]
    t, d = z.shape
    e = w.shape[0]

    @pl.when(step == 0)
    def _prep():
        ones_row = jnp.ones((8, d), jnp.float32)
        esq_ref[...] = jax.lax.dot_general(
            ones_row, w * w, (((1,), (1,)), ((), ())),
            precision=jax.lax.Precision.HIGHEST,
            preferred_element_type=jnp.float32)[0:1, :]

    zsq = jnp.sum(z * z, axis=1, keepdims=True)
    m2 = jax.lax.dot_general(
        (z + z).astype(jnp.bfloat16), w, (((1,), (1,)), ((), ())),
        preferred_element_type=jnp.float32)
    dist = (zsq + esq_ref[...]) - m2

    nwin = e // _WIN
    lanes_w = jax.lax.broadcasted_iota(jnp.int32, (1, _WIN), 1)
    mins, amins = [], []
    for k in range(nwin):
        dk = dist[:, k * _WIN:(k + 1) * _WIN]
        mk = jnp.min(dk, axis=1, keepdims=True)
        ak = jnp.min(jnp.where(dk == mk, lanes_w, _WIN), axis=1,
                     keepdims=True) + k * _WIN
        mins.append(mk)
        amins.append(ak)
    winval = mins[0]
    winidx = amins[0]
    run = mins[0].astype(jnp.bfloat16).astype(jnp.float32)
    for k in range(1, nwin):
        take = mins[k] < run
        winidx = jnp.where(take, amins[k], winidx)
        winval = jnp.where(take, mins[k], winval)
        run = jnp.where(take, mins[k].astype(jnp.bfloat16).astype(jnp.float32),
                        run)

    idx_ref[...] = winidx.reshape(1, 1, t)

    part = jnp.sum(winval, keepdims=True).reshape(1, 1)
    prev = jnp.where(step == 0, jnp.zeros((1, 1), jnp.float32), loss_ref[...])
    loss_ref[...] = prev + part


def _sc_gather_grouped(table128, idx4):
    n = idx4.shape[0]
    d = table128.shape[1]
    num_cores, num_subcores = 2, 16
    nw = num_cores * num_subcores
    b_per_w = n // nw
    chunk = 128
    nchunk = b_per_w // chunk
    mesh = plsc.VectorSubcoreMesh(core_axis_name="c", subcore_axis_name="s",
                                  num_cores=num_cores,
                                  num_subcores=num_subcores)

    @functools.partial(
        pl.kernel, mesh=mesh,
        out_type=jax.ShapeDtypeStruct((n, d), jnp.float32),
        scratch_types=[
            pltpu.VMEM((chunk,), jnp.int32),
            pltpu.VMEM((chunk, d), jnp.float32),
            pltpu.SemaphoreType.DMA,
        ],
    )
    def k(table_hbm, idx_hbm, out_hbm, idx_v, rows_v, sem):
        wid = jax.lax.axis_index("s") * num_cores + jax.lax.axis_index("c")
        base = wid * b_per_w
        for c in range(nchunk):
            off = base + c * chunk
            pltpu.sync_copy(idx_hbm.at[pl.ds(off, chunk)], idx_v)
            pltpu.async_copy(table_hbm.at[idx_v], rows_v, sem).wait()
            pltpu.sync_copy(rows_v, out_hbm.at[pl.ds(off, chunk)])

    return k(table128, idx4)


def _select_body(rows_ref, idx_ref, out_ref):
    rows = rows_ref[...]
    sub = idx_ref[...].reshape(-1, 1) & 3
    d = out_ref.shape[1]
    acc = jnp.where(sub == 0, rows[:, 0:d], 0.0)
    for q in range(1, 4):
        acc = jnp.where(sub == q, rows[:, q * d:(q + 1) * d], acc)
    out_ref[...] = acc


@functools.partial(jax.jit, static_argnames=())
def kernel(z, embedding_weight):
    b, s, d = z.shape
    n = b * s
    e = embedding_weight.shape[0]
    tile = 256
    g = n // tile
    zf = z.reshape(n, d)

    idx_out, loss_out = pl.pallas_call(
        _vq_idx_body,
        grid=(g,),
        in_specs=[
            pl.BlockSpec((tile, d), lambda i: (i, 0)),
            pl.BlockSpec((e, d), lambda i: (0, 0)),
        ],
        out_specs=[
            pl.BlockSpec((1, 1, tile), lambda i: (i, 0, 0)),
            pl.BlockSpec((1, 1), lambda i: (0, 0)),
        ],
        out_shape=[
            jax.ShapeDtypeStruct((g, 1, tile), jnp.int32),
            jax.ShapeDtypeStruct((1, 1), jnp.float32),
        ],
        scratch_shapes=[
            pltpu.VMEM((1, e), jnp.float32),
        ],
    )(zf, embedding_weight)

    indices_flat = idx_out.reshape(n)
    rows128 = _sc_gather_grouped(embedding_weight.reshape(e // 4, 4 * d),
                                 indices_flat >> 2)

    sel_tile = 1024
    g2 = n // sel_tile
    zq = pl.pallas_call(
        _select_body,
        grid=(g2,),
        in_specs=[
            pl.BlockSpec((sel_tile, 4 * d), lambda i: (i, 0)),
            pl.BlockSpec((1, 1, sel_tile), lambda i: (i, 0, 0)),
        ],
        out_specs=pl.BlockSpec((sel_tile, d), lambda i: (i, 0)),
        out_shape=jax.ShapeDtypeStruct((n, d), jnp.float32),
    )(rows128, indices_flat.reshape(g2, 1, sel_tile))

    z_q_st = zq.reshape(b, s, d)
    indices = indices_flat.reshape(b, s)
    mean = loss_out[0, 0] / jnp.float32(n * d)
    loss = mean + _COMMIT * mean
    return z_q_st, loss, indices


# speedup vs baseline: 3.7089x; 1.0501x over previous
import functools

import jax
import jax.numpy as jnp
from jax.experimental import pallas as pl
from jax.experimental.pallas import tpu as pltpu
from jax.experimental.pallas import tpu_sc as plsc

_COMMIT = 0.25
_WIN = 2048


def _vq_idx_body(zf_ref, w_ref, zsq_ref, esq_ref, idx_ref, loss_ref):
    step = pl.program_id(0)
    z = zf_ref[...]
    w = w_ref[...]
    t, d = z.shape
    e = w.shape[0]

    zsq = zsq_ref[...].reshape(t, 1)
    m2 = jax.lax.dot_general(
        (z + z).astype(jnp.bfloat16), w, (((1,), (1,)), ((), ())),
        preferred_element_type=jnp.float32)
    dist = (zsq + esq_ref[...].reshape(1, e)) - m2

    nwin = e // _WIN
    lanes_w = jax.lax.broadcasted_iota(jnp.int32, (1, _WIN), 1)
    mins, amins = [], []
    for k in range(nwin):
        dk = dist[:, k * _WIN:(k + 1) * _WIN]
        mk = jnp.min(dk, axis=1, keepdims=True)
        ak = jnp.min(jnp.where(dk == mk, lanes_w, _WIN), axis=1,
                     keepdims=True) + k * _WIN
        mins.append(mk)
        amins.append(ak)
    winval = mins[0]
    winidx = amins[0]
    run = mins[0].astype(jnp.bfloat16).astype(jnp.float32)
    for k in range(1, nwin):
        take = mins[k] < run
        winidx = jnp.where(take, amins[k], winidx)
        winval = jnp.where(take, mins[k], winval)
        run = jnp.where(take, mins[k].astype(jnp.bfloat16).astype(jnp.float32),
                        run)

    idx_ref[...] = winidx.reshape(1, 1, t)

    part = jnp.sum(winval, keepdims=True).reshape(1, 1)
    prev = jnp.where(step == 0, jnp.zeros((1, 1), jnp.float32), loss_ref[...])
    loss_ref[...] = prev + part


def _sc_gather_grouped(table128, idx4):
    n = idx4.shape[0]
    d = table128.shape[1]
    num_cores, num_subcores = 2, 16
    nw = num_cores * num_subcores
    b_per_w = n // nw
    chunk = 128
    nchunk = b_per_w // chunk
    mesh = plsc.VectorSubcoreMesh(core_axis_name="c", subcore_axis_name="s",
                                  num_cores=num_cores,
                                  num_subcores=num_subcores)

    @functools.partial(
        pl.kernel, mesh=mesh,
        out_type=jax.ShapeDtypeStruct((n, d), jnp.float32),
        scratch_types=[
            pltpu.VMEM((chunk,), jnp.int32),
            pltpu.VMEM((chunk, d), jnp.float32),
            pltpu.SemaphoreType.DMA,
        ],
    )
    def k(table_hbm, idx_hbm, out_hbm, idx_v, rows_v, sem):
        wid = jax.lax.axis_index("s") * num_cores + jax.lax.axis_index("c")
        base = wid * b_per_w
        for c in range(nchunk):
            off = base + c * chunk
            pltpu.sync_copy(idx_hbm.at[pl.ds(off, chunk)], idx_v)
            pltpu.async_copy(table_hbm.at[idx_v], rows_v, sem).wait()
            pltpu.sync_copy(rows_v, out_hbm.at[pl.ds(off, chunk)])

    return k(table128, idx4)


def _select_body(rows_ref, idx_ref, out_ref):
    rows = rows_ref[...]
    sub = idx_ref[...].reshape(-1, 1) & 3
    d = out_ref.shape[1]
    acc = jnp.where(sub == 0, rows[:, 0:d], 0.0)
    for q in range(1, 4):
        acc = jnp.where(sub == q, rows[:, q * d:(q + 1) * d], acc)
    out_ref[...] = acc


@functools.partial(jax.jit, static_argnames=())
def kernel(z, embedding_weight):
    b, s, d = z.shape
    n = b * s
    e = embedding_weight.shape[0]
    tile = 256
    g = n // tile
    zf = z.reshape(n, d)

    zsq_o = jnp.sum(zf ** 2, axis=1, keepdims=True)
    esq_o = jnp.sum(embedding_weight ** 2, axis=1).reshape(1, e)

    idx_out, loss_out = pl.pallas_call(
        _vq_idx_body,
        grid=(g,),
        in_specs=[
            pl.BlockSpec((tile, d), lambda i: (i, 0)),
            pl.BlockSpec((e, d), lambda i: (0, 0)),
            pl.BlockSpec((tile, 1), lambda i: (i, 0)),
            pl.BlockSpec((1, e), lambda i: (0, 0)),
        ],
        out_specs=[
            pl.BlockSpec((1, 1, tile), lambda i: (i, 0, 0)),
            pl.BlockSpec((1, 1), lambda i: (0, 0)),
        ],
        out_shape=[
            jax.ShapeDtypeStruct((g, 1, tile), jnp.int32),
            jax.ShapeDtypeStruct((1, 1), jnp.float32),
        ],
    )(zf, embedding_weight, zsq_o, esq_o)

    indices_flat = idx_out.reshape(n)
    rows128 = _sc_gather_grouped(embedding_weight.reshape(e // 4, 4 * d),
                                 indices_flat >> 2)

    sel_tile = 1024
    g2 = n // sel_tile
    zq = pl.pallas_call(
        _select_body,
        grid=(g2,),
        in_specs=[
            pl.BlockSpec((sel_tile, 4 * d), lambda i: (i, 0)),
            pl.BlockSpec((1, 1, sel_tile), lambda i: (i, 0, 0)),
        ],
        out_specs=pl.BlockSpec((sel_tile, d), lambda i: (i, 0)),
        out_shape=jax.ShapeDtypeStruct((n, d), jnp.float32),
    )(rows128, indices_flat.reshape(g2, 1, sel_tile))

    z_q_st = zq.reshape(b, s, d)
    indices = indices_flat.reshape(b, s)
    mean = loss_out[0, 0] / jnp.float32(n * d)
    loss = mean + _COMMIT * mean
    return z_q_st, loss, indices


# speedup vs baseline: 3.8746x; 1.0447x over previous
import functools

import jax
import jax.numpy as jnp
from jax.experimental import pallas as pl
from jax.experimental.pallas import tpu as pltpu
from jax.experimental.pallas import tpu_sc as plsc

_COMMIT = 0.25
_WIN = 2048


def _vq_idx_body(zf_ref, w_ref, zsq_ref, esq_ref, idx_ref, loss_ref):
    step = pl.program_id(0)
    z = zf_ref[...]
    w = w_ref[...]
    t, d = z.shape
    e = w.shape[0]

    zsq = zsq_ref[...].reshape(t, 1)
    m2 = jax.lax.dot_general(
        (z + z).astype(jnp.bfloat16), w, (((1,), (1,)), ((), ())),
        preferred_element_type=jnp.float32)
    dist = (zsq + esq_ref[...].reshape(1, e)) - m2

    nwin = e // _WIN
    lanes_w = jax.lax.broadcasted_iota(jnp.int32, (1, _WIN), 1)
    mins, amins = [], []
    for k in range(nwin):
        dk = dist[:, k * _WIN:(k + 1) * _WIN]
        mk = jnp.min(dk, axis=1, keepdims=True)
        ak = jnp.min(jnp.where(dk == mk, lanes_w, _WIN), axis=1,
                     keepdims=True) + k * _WIN
        mins.append(mk)
        amins.append(ak)
    winval = mins[0]
    winidx = amins[0]
    run = mins[0].astype(jnp.bfloat16).astype(jnp.float32)
    for k in range(1, nwin):
        take = mins[k] < run
        winidx = jnp.where(take, amins[k], winidx)
        winval = jnp.where(take, mins[k], winval)
        run = jnp.where(take, mins[k].astype(jnp.bfloat16).astype(jnp.float32),
                        run)

    idx_ref[...] = winidx.reshape(1, 1, t)

    part = jnp.sum(winval, keepdims=True).reshape(1, 1)
    prev = jnp.where(step == 0, jnp.zeros((1, 1), jnp.float32), loss_ref[...])
    loss_ref[...] = prev + part


def _sc_gather_grouped(table128, idx4):
    n = idx4.shape[0]
    d = table128.shape[1]
    num_cores, num_subcores = 2, 16
    nw = num_cores * num_subcores
    b_per_w = n // nw
    chunk = 128
    nchunk = b_per_w // chunk
    mesh = plsc.VectorSubcoreMesh(core_axis_name="c", subcore_axis_name="s",
                                  num_cores=num_cores,
                                  num_subcores=num_subcores)

    @functools.partial(
        pl.kernel, mesh=mesh,
        out_type=jax.ShapeDtypeStruct((n, d), jnp.float32),
        scratch_types=[
            pltpu.VMEM((chunk,), jnp.int32),
            pltpu.VMEM((chunk, d), jnp.float32),
            pltpu.SemaphoreType.DMA,
        ],
    )
    def k(table_hbm, idx_hbm, out_hbm, idx_v, rows_v, sem):
        wid = jax.lax.axis_index("s") * num_cores + jax.lax.axis_index("c")
        base = wid * b_per_w
        for c in range(nchunk):
            off = base + c * chunk
            pltpu.sync_copy(idx_hbm.at[pl.ds(off, chunk)], idx_v)
            pltpu.async_copy(table_hbm.at[idx_v], rows_v, sem).wait()
            pltpu.sync_copy(rows_v, out_hbm.at[pl.ds(off, chunk)])

    return k(table128, idx4)


def _select_body(rows_ref, idx_ref, out_ref):
    rows = rows_ref[...]
    sub = idx_ref[...].reshape(-1, 1) & 3
    d = out_ref.shape[1]
    acc = jnp.where(sub == 0, rows[:, 0:d], 0.0)
    for q in range(1, 4):
        acc = jnp.where(sub == q, rows[:, q * d:(q + 1) * d], acc)
    out_ref[...] = acc


@functools.partial(jax.jit, static_argnames=())
def kernel(z, embedding_weight):
    b, s, d = z.shape
    n = b * s
    e = embedding_weight.shape[0]
    tile = 512
    g = n // tile
    zf = z.reshape(n, d)

    zsq_o = jnp.sum(zf ** 2, axis=1, keepdims=True)
    esq_o = jnp.sum(embedding_weight ** 2, axis=1).reshape(1, e)

    idx_out, loss_out = pl.pallas_call(
        _vq_idx_body,
        grid=(g,),
        in_specs=[
            pl.BlockSpec((tile, d), lambda i: (i, 0)),
            pl.BlockSpec((e, d), lambda i: (0, 0)),
            pl.BlockSpec((tile, 1), lambda i: (i, 0)),
            pl.BlockSpec((1, e), lambda i: (0, 0)),
        ],
        out_specs=[
            pl.BlockSpec((1, 1, tile), lambda i: (i, 0, 0)),
            pl.BlockSpec((1, 1), lambda i: (0, 0)),
        ],
        out_shape=[
            jax.ShapeDtypeStruct((g, 1, tile), jnp.int32),
            jax.ShapeDtypeStruct((1, 1), jnp.float32),
        ],
    )(zf, embedding_weight, zsq_o, esq_o)

    indices_flat = idx_out.reshape(n)
    rows128 = _sc_gather_grouped(embedding_weight.reshape(e // 4, 4 * d),
                                 indices_flat >> 2)

    sel_tile = 1024
    g2 = n // sel_tile
    zq = pl.pallas_call(
        _select_body,
        grid=(g2,),
        in_specs=[
            pl.BlockSpec((sel_tile, 4 * d), lambda i: (i, 0)),
            pl.BlockSpec((1, 1, sel_tile), lambda i: (i, 0, 0)),
        ],
        out_specs=pl.BlockSpec((sel_tile, d), lambda i: (i, 0)),
        out_shape=jax.ShapeDtypeStruct((n, d), jnp.float32),
    )(rows128, indices_flat.reshape(g2, 1, sel_tile))

    z_q_st = zq.reshape(b, s, d)
    indices = indices_flat.reshape(b, s)
    mean = loss_out[0, 0] / jnp.float32(n * d)
    loss = mean + _COMMIT * mean
    return z_q_st, loss, indices
